# Initial kernel scaffold; baseline (speedup 1.0000x reference)
import jax
import jax.numpy as jnp
from jax.experimental import pallas as pl


def kernel(feats, edge_index, W0, b0, W1, b1):
    raise NotImplementedError("write your pallas kernel here")


# speedup vs baseline: 6.0129x; 6.0129x over previous
import functools

import jax
import jax.numpy as jnp
from jax import lax
from jax.experimental import pallas as pl
from jax.experimental.pallas import tpu as pltpu
from jax.experimental.pallas import tpu_sc as plsc

N = 10000
E = 320000
D_IN = 128
D_HID = 64
D_OUT = 128

NC = 2
NS = 16
NW = NC * NS
CB = 128
K = 79
EPT = K * CB
EP = NW * EPT
NP = 10240
RPT = NP // NS
NZC = RPT // CB

_f32 = jnp.float32


def _sc_mesh():
    return plsc.VectorSubcoreMesh(core_axis_name="c", subcore_axis_name="s")


def _deg_body(src3, dst3, out_o, out_i, sidx, didx, vals, acc_o, acc_i):
    cid = lax.axis_index("c")
    sid = lax.axis_index("s")
    wid = cid * NS + sid

    z16 = jnp.zeros((16,), _f32)
    o16 = jnp.ones((16,), _f32)

    def zfill(i, _):
        vals[i, :] = z16
        return 0

    lax.fori_loop(0, CB, zfill, 0)
    for k in range(NZC):
        r0 = sid * RPT + k * CB
        pltpu.sync_copy(vals, acc_o.at[pl.ds(r0, CB)])
        pltpu.sync_copy(vals, acc_i.at[pl.ds(r0, CB)])

    def ofill(i, _):
        vals[i, :] = o16
        return 0

    lax.fori_loop(0, CB, ofill, 0)
    pltpu.sync_copy(src3.at[wid], sidx)
    pltpu.sync_copy(dst3.at[wid], didx)
    plsc.subcore_barrier()

    def step(j, _):
        pltpu.sync_copy(vals, acc_o.at[sidx.at[j]], add=True)
        pltpu.sync_copy(vals, acc_i.at[didx.at[j]], add=True)
        return 0

    lax.fori_loop(0, K, step, 0)
    plsc.subcore_barrier()
    for k in range(NZC):
        r0 = sid * RPT + k * CB
        pltpu.sync_copy(acc_o.at[pl.ds(r0, CB)], out_o.at[cid, pl.ds(r0, CB)])
        pltpu.sync_copy(acc_i.at[pl.ds(r0, CB)], out_i.at[cid, pl.ds(r0, CB)])


def _sc_degrees(src3, dst3):
    kfn = pl.kernel(
        _deg_body,
        out_type=(
            jax.ShapeDtypeStruct((NC, NP, 16), _f32),
            jax.ShapeDtypeStruct((NC, NP, 16), _f32),
        ),
        mesh=_sc_mesh(),
        compiler_params=pltpu.CompilerParams(use_tc_tiling_on_sc=False),
        scratch_types=[
            pltpu.VMEM((K, CB), jnp.int32),
            pltpu.VMEM((K, CB), jnp.int32),
            pltpu.VMEM((CB, 16), _f32),
            pltpu.VMEM_SHARED((NP, 16), _f32),
            pltpu.VMEM_SHARED((NP, 16), _f32),
        ],
    )
    return kfn(src3, dst3)


def _edge_body(table, src3, dst3, out, sidx, didx, rows, acc):
    cid = lax.axis_index("c")
    sid = lax.axis_index("s")
    wid = cid * NS + sid

    z16 = jnp.zeros((16,), _f32)

    def zrow(i, _):
        for c in range(D_HID // 16):
            rows[i, pl.ds(c * 16, 16)] = z16
        return 0

    lax.fori_loop(0, CB, zrow, 0)
    for k in range(NZC):
        pltpu.sync_copy(rows, acc.at[pl.ds(sid * RPT + k * CB, CB)])
    pltpu.sync_copy(src3.at[wid], sidx)
    pltpu.sync_copy(dst3.at[wid], didx)
    plsc.subcore_barrier()

    def step(j, _):
        pltpu.sync_copy(table.at[sidx.at[j]], rows)
        pltpu.sync_copy(rows, acc.at[didx.at[j]], add=True)
        return 0

    lax.fori_loop(0, K, step, 0)
    plsc.subcore_barrier()
    for k in range(NZC):
        r0 = sid * RPT + k * CB
        pltpu.sync_copy(acc.at[pl.ds(r0, CB)], out.at[cid, pl.ds(r0, CB)])


def _sc_edge(table, src3, dst3):
    kfn = pl.kernel(
        _edge_body,
        out_type=jax.ShapeDtypeStruct((NC, NP, D_HID), _f32),
        mesh=_sc_mesh(),
        compiler_params=pltpu.CompilerParams(use_tc_tiling_on_sc=False),
        scratch_types=[
            pltpu.VMEM((K, CB), jnp.int32),
            pltpu.VMEM((K, CB), jnp.int32),
            pltpu.VMEM((CB, D_HID), _f32),
            pltpu.VMEM_SHARED((NP, D_HID), _f32),
        ],
    )
    return kfn(table, src3, dst3)


def _tc_premix(featsp, W0, do0, do1):
    def body(f, w, d0, d1, o):
        deg = d0[...][:, 0] + d1[...][:, 0]
        s = lax.rsqrt(jnp.maximum(deg, 1.0))
        o[...] = jnp.dot(f[...] * s[:, None], w[...],
                         preferred_element_type=_f32)

    return pl.pallas_call(
        body, out_shape=jax.ShapeDtypeStruct((NP, D_HID), _f32)
    )(featsp, W0, do0, do1)


def _tc_mid(a0, a1, di0, di1, do0, do1, b0):
    def body(a0r, a1r, di0r, di1r, do0r, do1r, br, h_ref, t_ref):
        s_in = lax.rsqrt(jnp.maximum(di0r[...][:, 0] + di1r[...][:, 0], 1.0))
        s_out = lax.rsqrt(jnp.maximum(do0r[...][:, 0] + do1r[...][:, 0], 1.0))
        agg = a0r[...] + a1r[...]
        h = jnp.maximum(agg * s_in[:, None] + br[...], 0.0)
        h_ref[...] = h
        row = lax.broadcasted_iota(jnp.int32, (NP, 1), 0)
        t_ref[...] = jnp.where(row < N, h * s_out[:, None], 0.0)

    return pl.pallas_call(
        body,
        out_shape=(
            jax.ShapeDtypeStruct((NP, D_HID), _f32),
            jax.ShapeDtypeStruct((NP, D_HID), _f32),
        ),
    )(a0, a1, di0, di1, do0, do1, b0)


def _tc_out(a0, a1, di0, di1, W1, b1):
    def body(a0r, a1r, di0r, di1r, wr, br, o_ref):
        s_in = lax.rsqrt(jnp.maximum(di0r[...][:, 0] + di1r[...][:, 0], 1.0))
        agg = (a0r[...] + a1r[...]) * s_in[:, None]
        o_ref[...] = jnp.dot(agg, wr[...], preferred_element_type=_f32) + br[...]

    return pl.pallas_call(
        body, out_shape=jax.ShapeDtypeStruct((NP, D_OUT), _f32)
    )(a0, a1, di0, di1, W1, b1)


def kernel(feats, edge_index, W0, b0, W1, b1):
    src = edge_index[0]
    dst = edge_index[1]
    pad = jnp.full((EP - E,), N, jnp.int32)
    src3 = jnp.concatenate([src, pad]).reshape(NW, K, CB)
    dst3 = jnp.concatenate([dst, pad]).reshape(NW, K, CB)
    featsp = jnp.pad(feats, ((0, NP - N), (0, 0)))

    dpo, dpi = _sc_degrees(src3, dst3)
    do0, do1 = dpo[0], dpo[1]
    di0, di1 = dpi[0], dpi[1]

    x0 = _tc_premix(featsp, W0, do0, do1)
    a0 = _sc_edge(x0, src3, dst3)
    hemb, t = _tc_mid(a0[0], a0[1], di0, di1, do0, do1,
                      b0.reshape(1, D_HID))
    a1 = _sc_edge(t, src3, dst3)
    out = _tc_out(a1[0], a1[1], di0, di1, W1, b1.reshape(1, D_OUT))
    return (hemb[:N], out[:N])
